# baseline (speedup 1.0000x reference)
import jax
import jax.numpy as jnp
from jax.experimental import pallas as pl
from jax.experimental.pallas import tpu as pltpu


def _copy_body(in_ref, out_ref, sem):
    copy = pltpu.make_async_copy(in_ref, out_ref, sem)
    copy.start()
    copy.wait()


def kernel(image_token, text_cls, topk_idx, selected_pooled, is_rare, strength):
    return pl.pallas_call(
        _copy_body,
        out_shape=jax.ShapeDtypeStruct(image_token.shape, image_token.dtype),
        in_specs=[pl.BlockSpec(memory_space=pl.ANY)],
        out_specs=pl.BlockSpec(memory_space=pl.ANY),
        scratch_shapes=[pltpu.SemaphoreType.DMA],
    )(image_token)


# speedup vs baseline: 1.0001x; 1.0001x over previous
import jax
import jax.numpy as jnp
from jax.experimental import pallas as pl
from jax.experimental.pallas import tpu as pltpu


_N_CHUNKS = 16


def _copy_body(in_ref, out_ref, *sems):
    copies = [
        pltpu.make_async_copy(in_ref.at[i], out_ref.at[i], sems[i])
        for i in range(_N_CHUNKS)
    ]
    for c in copies:
        c.start()
    for c in copies:
        c.wait()


def kernel(image_token, text_cls, topk_idx, selected_pooled, is_rare, strength):
    B, N, D = image_token.shape
    x = image_token.reshape(_N_CHUNKS, (B * N) // _N_CHUNKS, D)
    out = pl.pallas_call(
        _copy_body,
        out_shape=jax.ShapeDtypeStruct(x.shape, x.dtype),
        in_specs=[pl.BlockSpec(memory_space=pl.ANY)],
        out_specs=pl.BlockSpec(memory_space=pl.ANY),
        scratch_shapes=[pltpu.SemaphoreType.DMA] * _N_CHUNKS,
    )(x)
    return out.reshape(B, N, D)


# speedup vs baseline: 46.8067x; 46.7998x over previous
import jax
import jax.numpy as jnp
from jax.experimental import pallas as pl
from jax.experimental.pallas import tpu as pltpu


_BLOCK_ROWS = 1024


def _copy_body(in_ref, out_ref):
    out_ref[...] = in_ref[...]


def kernel(image_token, text_cls, topk_idx, selected_pooled, is_rare, strength):
    B, N, D = image_token.shape
    rows = B * N
    x = image_token.reshape(rows, D)
    out = pl.pallas_call(
        _copy_body,
        grid=(rows // _BLOCK_ROWS,),
        in_specs=[pl.BlockSpec((_BLOCK_ROWS, D), lambda i: (i, 0))],
        out_specs=pl.BlockSpec((_BLOCK_ROWS, D), lambda i: (i, 0)),
        out_shape=jax.ShapeDtypeStruct((rows, D), x.dtype),
    )(x)
    return out.reshape(B, N, D)


# speedup vs baseline: 46.9008x; 1.0020x over previous
import jax
import jax.numpy as jnp
from jax.experimental import pallas as pl
from jax.experimental.pallas import tpu as pltpu


_BLOCK_ROWS = 1024


def _copy_body(in_ref, out_ref):
    out_ref[...] = in_ref[...]


def kernel(image_token, text_cls, topk_idx, selected_pooled, is_rare, strength):
    B, N, D = image_token.shape
    rows = B * N
    x = image_token.reshape(rows, D)
    out = pl.pallas_call(
        _copy_body,
        grid=(rows // _BLOCK_ROWS,),
        in_specs=[pl.BlockSpec((_BLOCK_ROWS, D), lambda i: (i, 0))],
        out_specs=pl.BlockSpec((_BLOCK_ROWS, D), lambda i: (i, 0)),
        out_shape=jax.ShapeDtypeStruct((rows, D), x.dtype),
        compiler_params=pltpu.CompilerParams(
            dimension_semantics=("parallel",),
        ),
    )(x)
    return out.reshape(B, N, D)
